# baseline (speedup 1.0000x reference)
import functools

import numpy as np
import jax
import jax.numpy as jnp
from jax import lax
from jax.experimental import pallas as pl
from jax.experimental.pallas import tpu as pltpu
from jax.experimental.pallas import tpu_sc as plsc

_N = 100000
_S = 100
_B = 64
_CB = 2048
_J = 49
_NPAD = _J * _CB
_SPAD = 112

_KA = (np.uint32(1832780943), np.uint32(270669613))
_KB = (np.uint32(64467757), np.uint32(2916123636))

_TINY = np.float32(np.finfo(np.float32).tiny)


def _rotl(x, r):
    return (x << r) | (x >> (32 - r))


def _threefry_mix(x0, x1, rots):
    for r in rots:
        x0 = x0 + x1
        x1 = _rotl(x1, r)
        x1 = x1 ^ x0
    return x0, x1


def _threefry_bits(k1, k2, lo, key_folded=False):
    ks2 = np.uint32(k1 ^ k2 ^ np.uint32(0x1BD11BDA))
    r0 = (13, 15, 26, 6)
    r1 = (17, 29, 16, 24)
    x0 = jnp.full_like(lo, k1)
    x1 = lo if key_folded else lo + k2
    x0, x1 = _threefry_mix(x0, x1, r0)
    x0 = x0 + k2
    x1 = x1 + np.uint32(ks2 + np.uint32(1))
    x0, x1 = _threefry_mix(x0, x1, r1)
    x0 = x0 + ks2
    x1 = x1 + np.uint32(k1 + np.uint32(2))
    x0, x1 = _threefry_mix(x0, x1, r0)
    x0 = x0 + k1
    x1 = x1 + np.uint32(k2 + np.uint32(3))
    x0, x1 = _threefry_mix(x0, x1, r1)
    x0 = x0 + k2
    x1 = x1 + np.uint32(ks2 + np.uint32(4))
    x0, x1 = _threefry_mix(x0, x1, r0)
    x0 = x0 + ks2
    x1 = x1 + np.uint32(k1 + np.uint32(5))
    return x0 ^ x1


def _gumbel_from_bits(bits, exact=True):
    fb = lax.bitcast_convert_type(
        (bits >> np.uint32(9)) | np.uint32(0x3F800000), jnp.float32)
    f = fb - jnp.float32(1.0)
    if exact:
        f = jnp.maximum(_TINY, f * jnp.float32(1.0) + _TINY)
        return -jnp.log(-jnp.log(f))
    nln2 = jnp.float32(-0.6931471805599453)
    w = jnp.log2(f) * nln2
    return jnp.log2(w) * nln2


def _make_sampler(keys, s_count, b, n, cb, jblocks, interpret=False):
    (ka1, ka2), (kb1, kb2) = keys
    bn = np.uint32(b * n)

    def argstep(val, j, ju, lane_i, vbest, ibest, cbv):
        m = jnp.max(val, axis=1, keepdims=True)
        amask = val == m
        col = jnp.min(jnp.where(amask, lane_i, np.int32(2 ** 30)),
                      axis=1, keepdims=True)
        gidx = col + lax.convert_element_type(ju * np.uint32(cb), jnp.int32)
        upd = m > vbest
        return jnp.where(upd, m, vbest), jnp.where(upd, gidx, ibest)

    def body(la_ref, lb_ref, ia_ref, lwa_ref, ib_ref, lwb_ref, base_scr):
        s = pl.program_id(0)
        su = lax.convert_element_type(s, jnp.uint32)
        lane = lax.broadcasted_iota(jnp.uint32, (b, cb), 1)
        rowi = lax.broadcasted_iota(jnp.uint32, (b, cb), 0)
        base_scr[...] = rowi * np.uint32(n) + lane + su * bn

        def jbody(j, carry):
            va, ia, vb, ib = carry
            ju = lax.convert_element_type(j, jnp.uint32)
            ctr = base_scr[...] + ju * np.uint32(cb)
            lane_i = lax.broadcasted_iota(jnp.int32, (b, cb), 1)
            val_a = _gumbel_from_bits(
                _threefry_bits(ka1, ka2, ctr), exact=False) + la_ref[j]
            val_b = _gumbel_from_bits(
                _threefry_bits(kb1, kb2, ctr), exact=False) + lb_ref[j]
            va, ia = argstep(val_a, j, ju, lane_i, va, ia, cb)
            vb, ib = argstep(val_b, j, ju, lane_i, vb, ib, cb)
            return va, ia, vb, ib

        neg = jnp.full((b, 1), -jnp.inf, jnp.float32)
        zer = jnp.zeros((b, 1), jnp.int32)
        ma, ia, mb, ib = lax.fori_loop(0, jblocks, jbody,
                                       (neg, zer, neg, zer))
        rowc = lax.broadcasted_iota(jnp.uint32, (b, 1), 0)
        rb = rowc * np.uint32(n) + su * bn
        ctr_wa = rb + lax.convert_element_type(ia, jnp.uint32)
        ctr_wb = rb + lax.convert_element_type(ib, jnp.uint32)
        g_wa = _gumbel_from_bits(_threefry_bits(ka1, ka2, ctr_wa))
        g_wb = _gumbel_from_bits(_threefry_bits(kb1, kb2, ctr_wb))
        ia_ref[...] = ia.reshape(1, b, 1)
        lwa_ref[...] = (ma - g_wa).reshape(1, b, 1)
        ib_ref[...] = ib.reshape(1, b, 1)
        lwb_ref[...] = (mb - g_wb).reshape(1, b, 1)

    outspec = pl.BlockSpec((1, b, 1), lambda s: (s, 0, 0))
    return pl.pallas_call(
        body,
        grid=(s_count,),
        in_specs=[pl.BlockSpec((jblocks, b, cb), lambda s: (0, 0, 0)),
                  pl.BlockSpec((jblocks, b, cb), lambda s: (0, 0, 0))],
        out_specs=[outspec, outspec, outspec, outspec],
        out_shape=[jax.ShapeDtypeStruct((s_count, b, 1), jnp.int32),
                   jax.ShapeDtypeStruct((s_count, b, 1), jnp.float32),
                   jax.ShapeDtypeStruct((s_count, b, 1), jnp.int32),
                   jax.ShapeDtypeStruct((s_count, b, 1), jnp.float32)],
        scratch_shapes=[pltpu.VMEM((b, cb), jnp.uint32)],
        compiler_params=pltpu.CompilerParams(
            dimension_semantics=("parallel",)),
        interpret=interpret,
    )


_NW = 32
_ROWS_PER_W = _B // _NW
_L = 16


def _phase2_body(ia_hbm, ib_hbm, lwa_hbm, lwb_hbm, out_hbm,
                 iav, ibv, lav, lbv, rv, vv, row):
    wid = lax.axis_index("s") * 2 + lax.axis_index("c")
    for t in range(_ROWS_PER_W):
        brow = wid * _ROWS_PER_W + t
        pltpu.sync_copy(ia_hbm.at[brow], iav)
        pltpu.sync_copy(ib_hbm.at[brow], ibv)
        pltpu.sync_copy(lwa_hbm.at[brow], lav)
        pltpu.sync_copy(lwb_hbm.at[brow], lbv)

        nv = jnp.zeros((_L,), jnp.float32)
        for c in range(_SPAD // _L):
            sl = pl.ds(c * _L, _L)
            pa = jnp.exp(lav[sl]) - jnp.float32(1e-12)
            pb = jnp.exp(lbv[sl]) - jnp.float32(1e-12)
            v = pa * pb
            if (c + 1) * _L > _S:
                lane = lax.broadcasted_iota(jnp.int32, (_L,), 0) + np.int32(c * _L)
                v = jnp.where(lane < np.int32(_S), v, jnp.float32(0.0))
            r = lax.rem(iav[sl] + ibv[sl], np.int32(_N))
            rv[sl] = r
            vv[sl] = v
            nv = nv + v
        norm = nv[0]
        for k in range(1, _L):
            norm = norm + nv[k]
        norm = jnp.maximum(norm, jnp.float32(1e-12))
        for c in range(_SPAD // _L):
            sl = pl.ds(c * _L, _L)
            vv[sl] = vv[sl] / norm

        def zbody(i, _):
            off = pl.multiple_of(i * _L, _L)
            row[pl.ds(off, _L)] = jnp.zeros((_L,), jnp.float32)
            return 0

        lax.fori_loop(0, _N // _L, zbody, 0)

        iota16 = lax.broadcasted_iota(jnp.int32, (_L,), 0)
        for c in range(_SPAD // _L):
            sl = pl.ds(c * _L, _L)
            r16 = rv[sl]
            v16 = vv[sl]
            for k in range(_L):
                if c * _L + k < _S:
                    plsc.addupdate_scatter(row, [r16], v16,
                                           mask=iota16 == np.int32(k))

        pltpu.sync_copy(row, out_hbm.at[brow])


@functools.cache
def _phase2():
    return pl.kernel(
        _phase2_body,
        mesh=plsc.VectorSubcoreMesh(core_axis_name="c", subcore_axis_name="s"),
        out_type=jax.ShapeDtypeStruct((_B, _N), jnp.float32),
        compiler_params=pltpu.CompilerParams(needs_layout_passes=False),
        scratch_types=[
            pltpu.VMEM((_SPAD,), jnp.int32),
            pltpu.VMEM((_SPAD,), jnp.int32),
            pltpu.VMEM((_SPAD,), jnp.float32),
            pltpu.VMEM((_SPAD,), jnp.float32),
            pltpu.VMEM((_SPAD,), jnp.int32),
            pltpu.VMEM((_SPAD,), jnp.float32),
            pltpu.VMEM((_N,), jnp.float32),
        ],
    )


def _to_blocks(logits):
    lp = jnp.pad(logits, ((0, 0), (0, _NPAD - _N)),
                 constant_values=-np.inf)
    return lp.reshape(_B, _J, _CB).transpose(1, 0, 2)


def kernel(probs_a, probs_b):
    la = jnp.log(probs_a + 1e-12)
    lb = jnp.log(probs_b + 1e-12)
    la3 = _to_blocks(la)
    lb3 = _to_blocks(lb)
    sampler = _make_sampler((_KA, _KB), _S, _B, _N, _CB, _J)
    ia, lwa, ib, lwb = sampler(la3, lb3)
    pad = ((0, 0), (0, _SPAD - _S))
    ia = jnp.pad(ia[:, :, 0].T, pad)
    ib = jnp.pad(ib[:, :, 0].T, pad)
    lwa = jnp.pad(lwa[:, :, 0].T, pad)
    lwb = jnp.pad(lwb[:, :, 0].T, pad)
    out = _phase2()(ia, ib, lwa, lwb)
    return out


# speedup vs baseline: 1.0260x; 1.0260x over previous
import functools

import numpy as np
import jax
import jax.numpy as jnp
from jax import lax
from jax.experimental import pallas as pl
from jax.experimental.pallas import tpu as pltpu
from jax.experimental.pallas import tpu_sc as plsc

_N = 100000
_S = 100
_B = 64
_CB = 2048
_J = 49
_NPAD = _J * _CB
_SPAD = 112

_KA = (np.uint32(1832780943), np.uint32(270669613))
_KB = (np.uint32(64467757), np.uint32(2916123636))

_TINY = np.float32(np.finfo(np.float32).tiny)


def _rotl(x, r):
    return (x << r) | (x >> (32 - r))


def _threefry_mix(x0, x1, rots):
    for r in rots:
        x0 = x0 + x1
        x1 = _rotl(x1, r)
        x1 = x1 ^ x0
    return x0, x1


def _threefry_bits(k1, k2, lo, key_folded=False):
    ks2 = np.uint32(k1 ^ k2 ^ np.uint32(0x1BD11BDA))
    r0 = (13, 15, 26, 6)
    r1 = (17, 29, 16, 24)
    x0 = jnp.full_like(lo, k1)
    x1 = lo if key_folded else lo + k2
    x0, x1 = _threefry_mix(x0, x1, r0)
    x0 = x0 + k2
    x1 = x1 + np.uint32(ks2 + np.uint32(1))
    x0, x1 = _threefry_mix(x0, x1, r1)
    x0 = x0 + ks2
    x1 = x1 + np.uint32(k1 + np.uint32(2))
    x0, x1 = _threefry_mix(x0, x1, r0)
    x0 = x0 + k1
    x1 = x1 + np.uint32(k2 + np.uint32(3))
    x0, x1 = _threefry_mix(x0, x1, r1)
    x0 = x0 + k2
    x1 = x1 + np.uint32(ks2 + np.uint32(4))
    x0, x1 = _threefry_mix(x0, x1, r0)
    x0 = x0 + ks2
    x1 = x1 + np.uint32(k1 + np.uint32(5))
    return x0 ^ x1


def _gumbel_from_bits(bits, exact=True):
    fb = lax.bitcast_convert_type(
        (bits >> np.uint32(9)) | np.uint32(0x3F800000), jnp.float32)
    f = fb - jnp.float32(1.0)
    if exact:
        f = jnp.maximum(_TINY, f * jnp.float32(1.0) + _TINY)
    return -jnp.log(-jnp.log(f))


def _make_sampler(keys, s_count, b, n, cb, jblocks, interpret=False):
    (ka1, ka2), (kb1, kb2) = keys
    bn = np.uint32(b * n)

    def argstep(val, j, ju, lane_i, vbest, ibest, cbv):
        m = jnp.max(val, axis=1, keepdims=True)
        amask = val == m
        col = jnp.min(jnp.where(amask, lane_i, np.int32(2 ** 30)),
                      axis=1, keepdims=True)
        gidx = col + lax.convert_element_type(ju * np.uint32(cb), jnp.int32)
        upd = m > vbest
        return jnp.where(upd, m, vbest), jnp.where(upd, gidx, ibest)

    def body(la_ref, lb_ref, ia_ref, lwa_ref, ib_ref, lwb_ref, base_scr):
        s = pl.program_id(0)
        su = lax.convert_element_type(s, jnp.uint32)
        lane = lax.broadcasted_iota(jnp.uint32, (b, cb), 1)
        rowi = lax.broadcasted_iota(jnp.uint32, (b, cb), 0)
        base_scr[...] = rowi * np.uint32(n) + lane + su * bn

        def jbody(j, carry):
            va, ia, vb, ib = carry
            ju = lax.convert_element_type(j, jnp.uint32)
            ctr = base_scr[...] + ju * np.uint32(cb)
            lane_i = lax.broadcasted_iota(jnp.int32, (b, cb), 1)
            val_a = _gumbel_from_bits(
                _threefry_bits(ka1, ka2, ctr), exact=False) + la_ref[j]
            val_b = _gumbel_from_bits(
                _threefry_bits(kb1, kb2, ctr), exact=False) + lb_ref[j]
            va, ia = argstep(val_a, j, ju, lane_i, va, ia, cb)
            vb, ib = argstep(val_b, j, ju, lane_i, vb, ib, cb)
            return va, ia, vb, ib

        neg = jnp.full((b, 1), -jnp.inf, jnp.float32)
        zer = jnp.zeros((b, 1), jnp.int32)
        ma, ia, mb, ib = lax.fori_loop(0, jblocks, jbody,
                                       (neg, zer, neg, zer))
        rowc = lax.broadcasted_iota(jnp.uint32, (b, 1), 0)
        rb = rowc * np.uint32(n) + su * bn
        ctr_wa = rb + lax.convert_element_type(ia, jnp.uint32)
        ctr_wb = rb + lax.convert_element_type(ib, jnp.uint32)
        g_wa = _gumbel_from_bits(_threefry_bits(ka1, ka2, ctr_wa))
        g_wb = _gumbel_from_bits(_threefry_bits(kb1, kb2, ctr_wb))
        ia_ref[...] = ia.reshape(1, b, 1)
        lwa_ref[...] = (ma - g_wa).reshape(1, b, 1)
        ib_ref[...] = ib.reshape(1, b, 1)
        lwb_ref[...] = (mb - g_wb).reshape(1, b, 1)

    outspec = pl.BlockSpec((1, b, 1), lambda s: (s, 0, 0))
    return pl.pallas_call(
        body,
        grid=(s_count,),
        in_specs=[pl.BlockSpec((jblocks, b, cb), lambda s: (0, 0, 0)),
                  pl.BlockSpec((jblocks, b, cb), lambda s: (0, 0, 0))],
        out_specs=[outspec, outspec, outspec, outspec],
        out_shape=[jax.ShapeDtypeStruct((s_count, b, 1), jnp.int32),
                   jax.ShapeDtypeStruct((s_count, b, 1), jnp.float32),
                   jax.ShapeDtypeStruct((s_count, b, 1), jnp.int32),
                   jax.ShapeDtypeStruct((s_count, b, 1), jnp.float32)],
        scratch_shapes=[pltpu.VMEM((b, cb), jnp.uint32)],
        compiler_params=pltpu.CompilerParams(
            dimension_semantics=("parallel",)),
        interpret=interpret,
    )


_NW = 32
_ROWS_PER_W = _B // _NW
_L = 16


def _phase2_body(ia_hbm, ib_hbm, lwa_hbm, lwb_hbm, out_hbm,
                 iav, ibv, lav, lbv, rv, vv, row):
    wid = lax.axis_index("s") * 2 + lax.axis_index("c")
    for t in range(_ROWS_PER_W):
        brow = wid * _ROWS_PER_W + t
        pltpu.sync_copy(ia_hbm.at[brow], iav)
        pltpu.sync_copy(ib_hbm.at[brow], ibv)
        pltpu.sync_copy(lwa_hbm.at[brow], lav)
        pltpu.sync_copy(lwb_hbm.at[brow], lbv)

        nv = jnp.zeros((_L,), jnp.float32)
        for c in range(_SPAD // _L):
            sl = pl.ds(c * _L, _L)
            pa = jnp.exp(lav[sl]) - jnp.float32(1e-12)
            pb = jnp.exp(lbv[sl]) - jnp.float32(1e-12)
            v = pa * pb
            if (c + 1) * _L > _S:
                lane = lax.broadcasted_iota(jnp.int32, (_L,), 0) + np.int32(c * _L)
                v = jnp.where(lane < np.int32(_S), v, jnp.float32(0.0))
            r = lax.rem(iav[sl] + ibv[sl], np.int32(_N))
            rv[sl] = r
            vv[sl] = v
            nv = nv + v
        norm = nv[0]
        for k in range(1, _L):
            norm = norm + nv[k]
        norm = jnp.maximum(norm, jnp.float32(1e-12))
        for c in range(_SPAD // _L):
            sl = pl.ds(c * _L, _L)
            vv[sl] = vv[sl] / norm

        def zbody(i, _):
            off = pl.multiple_of(i * _L, _L)
            row[pl.ds(off, _L)] = jnp.zeros((_L,), jnp.float32)
            return 0

        lax.fori_loop(0, _N // _L, zbody, 0)

        iota16 = lax.broadcasted_iota(jnp.int32, (_L,), 0)
        for c in range(_SPAD // _L):
            sl = pl.ds(c * _L, _L)
            r16 = rv[sl]
            v16 = vv[sl]
            for k in range(_L):
                if c * _L + k < _S:
                    plsc.addupdate_scatter(row, [r16], v16,
                                           mask=iota16 == np.int32(k))

        pltpu.sync_copy(row, out_hbm.at[brow])


@functools.cache
def _phase2():
    return pl.kernel(
        _phase2_body,
        mesh=plsc.VectorSubcoreMesh(core_axis_name="c", subcore_axis_name="s"),
        out_type=jax.ShapeDtypeStruct((_B, _N), jnp.float32),
        compiler_params=pltpu.CompilerParams(needs_layout_passes=False),
        scratch_types=[
            pltpu.VMEM((_SPAD,), jnp.int32),
            pltpu.VMEM((_SPAD,), jnp.int32),
            pltpu.VMEM((_SPAD,), jnp.float32),
            pltpu.VMEM((_SPAD,), jnp.float32),
            pltpu.VMEM((_SPAD,), jnp.int32),
            pltpu.VMEM((_SPAD,), jnp.float32),
            pltpu.VMEM((_N,), jnp.float32),
        ],
    )


def _to_blocks(logits):
    lp = jnp.pad(logits, ((0, 0), (0, _NPAD - _N)),
                 constant_values=-np.inf)
    return lp.reshape(_B, _J, _CB).transpose(1, 0, 2)


def kernel(probs_a, probs_b):
    la = jnp.log(probs_a + 1e-12)
    lb = jnp.log(probs_b + 1e-12)
    la3 = _to_blocks(la)
    lb3 = _to_blocks(lb)
    sampler = _make_sampler((_KA, _KB), _S, _B, _N, _CB, _J)
    ia, lwa, ib, lwb = sampler(la3, lb3)
    pad = ((0, 0), (0, _SPAD - _S))
    ia = jnp.pad(ia[:, :, 0].T, pad)
    ib = jnp.pad(ib[:, :, 0].T, pad)
    lwa = jnp.pad(lwa[:, :, 0].T, pad)
    lwb = jnp.pad(lwb[:, :, 0].T, pad)
    out = _phase2()(ia, ib, lwa, lwb)
    return out


# speedup vs baseline: 1.0410x; 1.0146x over previous
import functools

import numpy as np
import jax
import jax.numpy as jnp
from jax import lax
from jax.experimental import pallas as pl
from jax.experimental.pallas import tpu as pltpu
from jax.experimental.pallas import tpu_sc as plsc

_N = 100000
_S = 100
_B = 64
_CB = 2048
_J = 49
_NPAD = _J * _CB
_SPAD = 112

_KA = (np.uint32(1832780943), np.uint32(270669613))
_KB = (np.uint32(64467757), np.uint32(2916123636))

_TINY = np.float32(np.finfo(np.float32).tiny)


def _rotl(x, r):
    return (x << r) | (x >> (32 - r))


def _threefry_mix(x0, x1, rots):
    for r in rots:
        x0 = x0 + x1
        x1 = _rotl(x1, r)
        x1 = x1 ^ x0
    return x0, x1


def _threefry_bits(k1, k2, lo, key_folded=False):
    ks2 = np.uint32(k1 ^ k2 ^ np.uint32(0x1BD11BDA))
    r0 = (13, 15, 26, 6)
    r1 = (17, 29, 16, 24)
    x0 = jnp.full_like(lo, k1)
    x1 = lo if key_folded else lo + k2
    x0, x1 = _threefry_mix(x0, x1, r0)
    x0 = x0 + k2
    x1 = x1 + np.uint32(ks2 + np.uint32(1))
    x0, x1 = _threefry_mix(x0, x1, r1)
    x0 = x0 + ks2
    x1 = x1 + np.uint32(k1 + np.uint32(2))
    x0, x1 = _threefry_mix(x0, x1, r0)
    x0 = x0 + k1
    x1 = x1 + np.uint32(k2 + np.uint32(3))
    x0, x1 = _threefry_mix(x0, x1, r1)
    x0 = x0 + k2
    x1 = x1 + np.uint32(ks2 + np.uint32(4))
    x0, x1 = _threefry_mix(x0, x1, r0)
    x0 = x0 + ks2
    x1 = x1 + np.uint32(k1 + np.uint32(5))
    return x0 ^ x1


def _gumbel_from_bits(bits, exact=True):
    fb = lax.bitcast_convert_type(
        (bits >> np.uint32(9)) | np.uint32(0x3F800000), jnp.float32)
    f = fb - jnp.float32(1.0)
    if exact:
        f = jnp.maximum(_TINY, f * jnp.float32(1.0) + _TINY)
    return -jnp.log(-jnp.log(f))


def _make_sampler(keys, s_count, b, n, cb, jblocks, interpret=False):
    (ka1, ka2), (kb1, kb2) = keys
    bn = np.uint32(b * n)

    def argstep(val, j, ju, lane_i, vbest, ibest, cbv):
        m = jnp.max(val, axis=1, keepdims=True)
        amask = val == m
        col = jnp.min(jnp.where(amask, lane_i, np.int32(2 ** 30)),
                      axis=1, keepdims=True)
        gidx = col + lax.convert_element_type(ju * np.uint32(cb), jnp.int32)
        upd = m > vbest
        return jnp.where(upd, m, vbest), jnp.where(upd, gidx, ibest)

    def body(la_ref, lb_ref, ia_ref, lwa_ref, ib_ref, lwb_ref, base_scr):
        s = pl.program_id(0)
        su = lax.convert_element_type(s, jnp.uint32)
        lane = lax.broadcasted_iota(jnp.uint32, (b, cb), 1)
        rowi = lax.broadcasted_iota(jnp.uint32, (b, cb), 0)
        base_scr[...] = rowi * np.uint32(n) + lane + su * bn

        def jbody(j, carry):
            va, ia, vb, ib = carry
            ju = lax.convert_element_type(j, jnp.uint32)
            ctr = base_scr[...] + ju * np.uint32(cb)
            lane_i = lax.broadcasted_iota(jnp.int32, (b, cb), 1)
            val_a = _gumbel_from_bits(
                _threefry_bits(ka1, ka2, ctr), exact=False) + la_ref[j]
            val_b = _gumbel_from_bits(
                _threefry_bits(kb1, kb2, ctr), exact=False) + lb_ref[j]
            va, ia = argstep(val_a, j, ju, lane_i, va, ia, cb)
            vb, ib = argstep(val_b, j, ju, lane_i, vb, ib, cb)
            return va, ia, vb, ib

        neg = jnp.full((b, 1), -jnp.inf, jnp.float32)
        zer = jnp.zeros((b, 1), jnp.int32)
        ma, ia, mb, ib = lax.fori_loop(0, jblocks, jbody,
                                       (neg, zer, neg, zer), unroll=2)
        rowc = lax.broadcasted_iota(jnp.uint32, (b, 1), 0)
        rb = rowc * np.uint32(n) + su * bn
        ctr_wa = rb + lax.convert_element_type(ia, jnp.uint32)
        ctr_wb = rb + lax.convert_element_type(ib, jnp.uint32)
        g_wa = _gumbel_from_bits(_threefry_bits(ka1, ka2, ctr_wa))
        g_wb = _gumbel_from_bits(_threefry_bits(kb1, kb2, ctr_wb))
        ia_ref[...] = ia.reshape(1, b, 1)
        lwa_ref[...] = (ma - g_wa).reshape(1, b, 1)
        ib_ref[...] = ib.reshape(1, b, 1)
        lwb_ref[...] = (mb - g_wb).reshape(1, b, 1)

    outspec = pl.BlockSpec((1, b, 1), lambda s: (s, 0, 0))
    return pl.pallas_call(
        body,
        grid=(s_count,),
        in_specs=[pl.BlockSpec((jblocks, b, cb), lambda s: (0, 0, 0)),
                  pl.BlockSpec((jblocks, b, cb), lambda s: (0, 0, 0))],
        out_specs=[outspec, outspec, outspec, outspec],
        out_shape=[jax.ShapeDtypeStruct((s_count, b, 1), jnp.int32),
                   jax.ShapeDtypeStruct((s_count, b, 1), jnp.float32),
                   jax.ShapeDtypeStruct((s_count, b, 1), jnp.int32),
                   jax.ShapeDtypeStruct((s_count, b, 1), jnp.float32)],
        scratch_shapes=[pltpu.VMEM((b, cb), jnp.uint32)],
        compiler_params=pltpu.CompilerParams(
            dimension_semantics=("parallel",)),
        interpret=interpret,
    )


_NW = 32
_ROWS_PER_W = _B // _NW
_L = 16


def _phase2_body(ia_hbm, ib_hbm, lwa_hbm, lwb_hbm, out_hbm,
                 iav, ibv, lav, lbv, rv, vv, row):
    wid = lax.axis_index("s") * 2 + lax.axis_index("c")
    for t in range(_ROWS_PER_W):
        brow = wid * _ROWS_PER_W + t
        pltpu.sync_copy(ia_hbm.at[brow], iav)
        pltpu.sync_copy(ib_hbm.at[brow], ibv)
        pltpu.sync_copy(lwa_hbm.at[brow], lav)
        pltpu.sync_copy(lwb_hbm.at[brow], lbv)

        nv = jnp.zeros((_L,), jnp.float32)
        for c in range(_SPAD // _L):
            sl = pl.ds(c * _L, _L)
            pa = jnp.exp(lav[sl]) - jnp.float32(1e-12)
            pb = jnp.exp(lbv[sl]) - jnp.float32(1e-12)
            v = pa * pb
            if (c + 1) * _L > _S:
                lane = lax.broadcasted_iota(jnp.int32, (_L,), 0) + np.int32(c * _L)
                v = jnp.where(lane < np.int32(_S), v, jnp.float32(0.0))
            r = lax.rem(iav[sl] + ibv[sl], np.int32(_N))
            rv[sl] = r
            vv[sl] = v
            nv = nv + v
        norm = nv[0]
        for k in range(1, _L):
            norm = norm + nv[k]
        norm = jnp.maximum(norm, jnp.float32(1e-12))
        for c in range(_SPAD // _L):
            sl = pl.ds(c * _L, _L)
            vv[sl] = vv[sl] / norm

        def zbody(i, _):
            off = pl.multiple_of(i * _L, _L)
            row[pl.ds(off, _L)] = jnp.zeros((_L,), jnp.float32)
            return 0

        lax.fori_loop(0, _N // _L, zbody, 0)

        iota16 = lax.broadcasted_iota(jnp.int32, (_L,), 0)
        for c in range(_SPAD // _L):
            sl = pl.ds(c * _L, _L)
            r16 = rv[sl]
            v16 = vv[sl]
            for k in range(_L):
                if c * _L + k < _S:
                    plsc.addupdate_scatter(row, [r16], v16,
                                           mask=iota16 == np.int32(k))

        pltpu.sync_copy(row, out_hbm.at[brow])


@functools.cache
def _phase2():
    return pl.kernel(
        _phase2_body,
        mesh=plsc.VectorSubcoreMesh(core_axis_name="c", subcore_axis_name="s"),
        out_type=jax.ShapeDtypeStruct((_B, _N), jnp.float32),
        compiler_params=pltpu.CompilerParams(needs_layout_passes=False),
        scratch_types=[
            pltpu.VMEM((_SPAD,), jnp.int32),
            pltpu.VMEM((_SPAD,), jnp.int32),
            pltpu.VMEM((_SPAD,), jnp.float32),
            pltpu.VMEM((_SPAD,), jnp.float32),
            pltpu.VMEM((_SPAD,), jnp.int32),
            pltpu.VMEM((_SPAD,), jnp.float32),
            pltpu.VMEM((_N,), jnp.float32),
        ],
    )


def _to_blocks(logits):
    lp = jnp.pad(logits, ((0, 0), (0, _NPAD - _N)),
                 constant_values=-np.inf)
    return lp.reshape(_B, _J, _CB).transpose(1, 0, 2)


def kernel(probs_a, probs_b):
    la = jnp.log(probs_a + 1e-12)
    lb = jnp.log(probs_b + 1e-12)
    la3 = _to_blocks(la)
    lb3 = _to_blocks(lb)
    sampler = _make_sampler((_KA, _KB), _S, _B, _N, _CB, _J)
    ia, lwa, ib, lwb = sampler(la3, lb3)
    pad = ((0, 0), (0, _SPAD - _S))
    ia = jnp.pad(ia[:, :, 0].T, pad)
    ib = jnp.pad(ib[:, :, 0].T, pad)
    lwa = jnp.pad(lwa[:, :, 0].T, pad)
    lwb = jnp.pad(lwb[:, :, 0].T, pad)
    out = _phase2()(ia, ib, lwa, lwb)
    return out


# speedup vs baseline: 1.0473x; 1.0061x over previous
import functools

import numpy as np
import jax
import jax.numpy as jnp
from jax import lax
from jax.experimental import pallas as pl
from jax.experimental.pallas import tpu as pltpu
from jax.experimental.pallas import tpu_sc as plsc

_N = 100000
_S = 100
_B = 64
_CB = 2048
_J = 49
_NPAD = _J * _CB
_SPAD = 112

_KA = (np.uint32(1832780943), np.uint32(270669613))
_KB = (np.uint32(64467757), np.uint32(2916123636))

_TINY = np.float32(np.finfo(np.float32).tiny)


def _rotl(x, r):
    return (x << r) | (x >> (32 - r))


def _threefry_mix(x0, x1, rots):
    for r in rots:
        x0 = x0 + x1
        x1 = _rotl(x1, r)
        x1 = x1 ^ x0
    return x0, x1


def _threefry_bits(k1, k2, lo, key_folded=False):
    ks2 = np.uint32(k1 ^ k2 ^ np.uint32(0x1BD11BDA))
    r0 = (13, 15, 26, 6)
    r1 = (17, 29, 16, 24)
    x0 = jnp.full_like(lo, k1)
    x1 = lo if key_folded else lo + k2
    x0, x1 = _threefry_mix(x0, x1, r0)
    x0 = x0 + k2
    x1 = x1 + np.uint32(ks2 + np.uint32(1))
    x0, x1 = _threefry_mix(x0, x1, r1)
    x0 = x0 + ks2
    x1 = x1 + np.uint32(k1 + np.uint32(2))
    x0, x1 = _threefry_mix(x0, x1, r0)
    x0 = x0 + k1
    x1 = x1 + np.uint32(k2 + np.uint32(3))
    x0, x1 = _threefry_mix(x0, x1, r1)
    x0 = x0 + k2
    x1 = x1 + np.uint32(ks2 + np.uint32(4))
    x0, x1 = _threefry_mix(x0, x1, r0)
    x0 = x0 + ks2
    x1 = x1 + np.uint32(k1 + np.uint32(5))
    return x0 ^ x1


def _gumbel_from_bits(bits, exact=True):
    fb = lax.bitcast_convert_type(
        (bits >> np.uint32(9)) | np.uint32(0x3F800000), jnp.float32)
    f = fb - jnp.float32(1.0)
    if exact:
        f = jnp.maximum(_TINY, f * jnp.float32(1.0) + _TINY)
    return -jnp.log(-jnp.log(f))


def _make_sampler(keys, s_count, b, n, cb, jblocks, interpret=False):
    (ka1, ka2), (kb1, kb2) = keys
    bn = np.uint32(b * n)

    def argstep(val, j, ju, lane_i, vbest, ibest, cbv):
        m = jnp.max(val, axis=1, keepdims=True)
        amask = val == m
        col = jnp.min(jnp.where(amask, lane_i, np.int32(2 ** 30)),
                      axis=1, keepdims=True)
        gidx = col + lax.convert_element_type(ju * np.uint32(cb), jnp.int32)
        upd = m > vbest
        return jnp.where(upd, m, vbest), jnp.where(upd, gidx, ibest)

    def body(la_ref, lb_ref, ia_ref, lwa_ref, ib_ref, lwb_ref, base_scr):
        s = pl.program_id(0)
        su = lax.convert_element_type(s, jnp.uint32)
        lane = lax.broadcasted_iota(jnp.uint32, (b, cb), 1)
        rowi = lax.broadcasted_iota(jnp.uint32, (b, cb), 0)
        base_scr[...] = rowi * np.uint32(n) + lane + su * bn

        def jbody(j, carry):
            va, ia, vb, ib = carry
            ju = lax.convert_element_type(j, jnp.uint32)
            ctr = base_scr[...] + ju * np.uint32(cb)
            lane_i = lax.broadcasted_iota(jnp.int32, (b, cb), 1)
            val_a = _gumbel_from_bits(
                _threefry_bits(ka1, ka2, ctr), exact=False) + la_ref[j]
            val_b = _gumbel_from_bits(
                _threefry_bits(kb1, kb2, ctr), exact=False) + lb_ref[j]
            va, ia = argstep(val_a, j, ju, lane_i, va, ia, cb)
            vb, ib = argstep(val_b, j, ju, lane_i, vb, ib, cb)
            return va, ia, vb, ib

        neg = jnp.full((b, 1), -jnp.inf, jnp.float32)
        zer = jnp.zeros((b, 1), jnp.int32)
        ma, ia, mb, ib = lax.fori_loop(0, jblocks, jbody,
                                       (neg, zer, neg, zer), unroll=4)
        rowc = lax.broadcasted_iota(jnp.uint32, (b, 1), 0)
        rb = rowc * np.uint32(n) + su * bn
        ctr_wa = rb + lax.convert_element_type(ia, jnp.uint32)
        ctr_wb = rb + lax.convert_element_type(ib, jnp.uint32)
        g_wa = _gumbel_from_bits(_threefry_bits(ka1, ka2, ctr_wa))
        g_wb = _gumbel_from_bits(_threefry_bits(kb1, kb2, ctr_wb))
        ia_ref[...] = ia.reshape(1, b, 1)
        lwa_ref[...] = (ma - g_wa).reshape(1, b, 1)
        ib_ref[...] = ib.reshape(1, b, 1)
        lwb_ref[...] = (mb - g_wb).reshape(1, b, 1)

    outspec = pl.BlockSpec((1, b, 1), lambda s: (s, 0, 0))
    return pl.pallas_call(
        body,
        grid=(s_count,),
        in_specs=[pl.BlockSpec((jblocks, b, cb), lambda s: (0, 0, 0)),
                  pl.BlockSpec((jblocks, b, cb), lambda s: (0, 0, 0))],
        out_specs=[outspec, outspec, outspec, outspec],
        out_shape=[jax.ShapeDtypeStruct((s_count, b, 1), jnp.int32),
                   jax.ShapeDtypeStruct((s_count, b, 1), jnp.float32),
                   jax.ShapeDtypeStruct((s_count, b, 1), jnp.int32),
                   jax.ShapeDtypeStruct((s_count, b, 1), jnp.float32)],
        scratch_shapes=[pltpu.VMEM((b, cb), jnp.uint32)],
        compiler_params=pltpu.CompilerParams(
            dimension_semantics=("parallel",)),
        interpret=interpret,
    )


_NW = 32
_ROWS_PER_W = _B // _NW
_L = 16


def _phase2_body(ia_hbm, ib_hbm, lwa_hbm, lwb_hbm, out_hbm,
                 iav, ibv, lav, lbv, rv, vv, row):
    wid = lax.axis_index("s") * 2 + lax.axis_index("c")
    for t in range(_ROWS_PER_W):
        brow = wid * _ROWS_PER_W + t
        pltpu.sync_copy(ia_hbm.at[brow], iav)
        pltpu.sync_copy(ib_hbm.at[brow], ibv)
        pltpu.sync_copy(lwa_hbm.at[brow], lav)
        pltpu.sync_copy(lwb_hbm.at[brow], lbv)

        nv = jnp.zeros((_L,), jnp.float32)
        for c in range(_SPAD // _L):
            sl = pl.ds(c * _L, _L)
            pa = jnp.exp(lav[sl]) - jnp.float32(1e-12)
            pb = jnp.exp(lbv[sl]) - jnp.float32(1e-12)
            v = pa * pb
            if (c + 1) * _L > _S:
                lane = lax.broadcasted_iota(jnp.int32, (_L,), 0) + np.int32(c * _L)
                v = jnp.where(lane < np.int32(_S), v, jnp.float32(0.0))
            r = lax.rem(iav[sl] + ibv[sl], np.int32(_N))
            rv[sl] = r
            vv[sl] = v
            nv = nv + v
        norm = nv[0]
        for k in range(1, _L):
            norm = norm + nv[k]
        norm = jnp.maximum(norm, jnp.float32(1e-12))
        for c in range(_SPAD // _L):
            sl = pl.ds(c * _L, _L)
            vv[sl] = vv[sl] / norm

        def zbody(i, _):
            off = pl.multiple_of(i * _L, _L)
            row[pl.ds(off, _L)] = jnp.zeros((_L,), jnp.float32)
            return 0

        lax.fori_loop(0, _N // _L, zbody, 0)

        iota16 = lax.broadcasted_iota(jnp.int32, (_L,), 0)
        for c in range(_SPAD // _L):
            sl = pl.ds(c * _L, _L)
            r16 = rv[sl]
            v16 = vv[sl]
            for k in range(_L):
                if c * _L + k < _S:
                    plsc.addupdate_scatter(row, [r16], v16,
                                           mask=iota16 == np.int32(k))

        pltpu.sync_copy(row, out_hbm.at[brow])


@functools.cache
def _phase2():
    return pl.kernel(
        _phase2_body,
        mesh=plsc.VectorSubcoreMesh(core_axis_name="c", subcore_axis_name="s"),
        out_type=jax.ShapeDtypeStruct((_B, _N), jnp.float32),
        compiler_params=pltpu.CompilerParams(needs_layout_passes=False),
        scratch_types=[
            pltpu.VMEM((_SPAD,), jnp.int32),
            pltpu.VMEM((_SPAD,), jnp.int32),
            pltpu.VMEM((_SPAD,), jnp.float32),
            pltpu.VMEM((_SPAD,), jnp.float32),
            pltpu.VMEM((_SPAD,), jnp.int32),
            pltpu.VMEM((_SPAD,), jnp.float32),
            pltpu.VMEM((_N,), jnp.float32),
        ],
    )


def _to_blocks(logits):
    lp = jnp.pad(logits, ((0, 0), (0, _NPAD - _N)),
                 constant_values=-np.inf)
    return lp.reshape(_B, _J, _CB).transpose(1, 0, 2)


def kernel(probs_a, probs_b):
    la = jnp.log(probs_a + 1e-12)
    lb = jnp.log(probs_b + 1e-12)
    la3 = _to_blocks(la)
    lb3 = _to_blocks(lb)
    sampler = _make_sampler((_KA, _KB), _S, _B, _N, _CB, _J)
    ia, lwa, ib, lwb = sampler(la3, lb3)
    pad = ((0, 0), (0, _SPAD - _S))
    ia = jnp.pad(ia[:, :, 0].T, pad)
    ib = jnp.pad(ib[:, :, 0].T, pad)
    lwa = jnp.pad(lwa[:, :, 0].T, pad)
    lwb = jnp.pad(lwb[:, :, 0].T, pad)
    out = _phase2()(ia, ib, lwa, lwb)
    return out


# speedup vs baseline: 1.0724x; 1.0240x over previous
import functools

import numpy as np
import jax
import jax.numpy as jnp
from jax import lax
from jax.experimental import pallas as pl
from jax.experimental.pallas import tpu as pltpu
from jax.experimental.pallas import tpu_sc as plsc

_N = 100000
_S = 100
_B = 64
_CB = 1024
_J = 98
_NPAD = _J * _CB
_SPAD = 112

_KA = (np.uint32(1832780943), np.uint32(270669613))
_KB = (np.uint32(64467757), np.uint32(2916123636))

_TINY = np.float32(np.finfo(np.float32).tiny)


def _rotl(x, r):
    return (x << r) | (x >> (32 - r))


def _threefry_mix(x0, x1, rots):
    for r in rots:
        x0 = x0 + x1
        x1 = _rotl(x1, r)
        x1 = x1 ^ x0
    return x0, x1


def _threefry_bits(k1, k2, lo, key_folded=False):
    ks2 = np.uint32(k1 ^ k2 ^ np.uint32(0x1BD11BDA))
    r0 = (13, 15, 26, 6)
    r1 = (17, 29, 16, 24)
    x0 = jnp.full_like(lo, k1)
    x1 = lo if key_folded else lo + k2
    x0, x1 = _threefry_mix(x0, x1, r0)
    x0 = x0 + k2
    x1 = x1 + np.uint32(ks2 + np.uint32(1))
    x0, x1 = _threefry_mix(x0, x1, r1)
    x0 = x0 + ks2
    x1 = x1 + np.uint32(k1 + np.uint32(2))
    x0, x1 = _threefry_mix(x0, x1, r0)
    x0 = x0 + k1
    x1 = x1 + np.uint32(k2 + np.uint32(3))
    x0, x1 = _threefry_mix(x0, x1, r1)
    x0 = x0 + k2
    x1 = x1 + np.uint32(ks2 + np.uint32(4))
    x0, x1 = _threefry_mix(x0, x1, r0)
    x0 = x0 + ks2
    x1 = x1 + np.uint32(k1 + np.uint32(5))
    return x0 ^ x1


def _gumbel_from_bits(bits, exact=True):
    fb = lax.bitcast_convert_type(
        (bits >> np.uint32(9)) | np.uint32(0x3F800000), jnp.float32)
    f = fb - jnp.float32(1.0)
    if exact:
        f = jnp.maximum(_TINY, f * jnp.float32(1.0) + _TINY)
    return -jnp.log(-jnp.log(f))


def _make_sampler(keys, s_count, b, n, cb, jblocks, interpret=False):
    (ka1, ka2), (kb1, kb2) = keys
    bn = np.uint32(b * n)

    def argstep(val, j, ju, lane_i, vbest, ibest, cbv):
        m = jnp.max(val, axis=1, keepdims=True)
        amask = val == m
        col = jnp.min(jnp.where(amask, lane_i, np.int32(2 ** 30)),
                      axis=1, keepdims=True)
        gidx = col + lax.convert_element_type(ju * np.uint32(cb), jnp.int32)
        upd = m > vbest
        return jnp.where(upd, m, vbest), jnp.where(upd, gidx, ibest)

    def body(la_ref, lb_ref, ia_ref, lwa_ref, ib_ref, lwb_ref, base_scr):
        s = pl.program_id(0)
        su = lax.convert_element_type(s, jnp.uint32)
        lane = lax.broadcasted_iota(jnp.uint32, (b, cb), 1)
        rowi = lax.broadcasted_iota(jnp.uint32, (b, cb), 0)
        base_scr[...] = rowi * np.uint32(n) + lane + su * bn

        def jbody(j, carry):
            va, ia, vb, ib = carry
            ju = lax.convert_element_type(j, jnp.uint32)
            ctr = base_scr[...] + ju * np.uint32(cb)
            lane_i = lax.broadcasted_iota(jnp.int32, (b, cb), 1)
            val_a = _gumbel_from_bits(
                _threefry_bits(ka1, ka2, ctr), exact=False) + la_ref[j]
            val_b = _gumbel_from_bits(
                _threefry_bits(kb1, kb2, ctr), exact=False) + lb_ref[j]
            va, ia = argstep(val_a, j, ju, lane_i, va, ia, cb)
            vb, ib = argstep(val_b, j, ju, lane_i, vb, ib, cb)
            return va, ia, vb, ib

        neg = jnp.full((b, 1), -jnp.inf, jnp.float32)
        zer = jnp.zeros((b, 1), jnp.int32)
        ma, ia, mb, ib = lax.fori_loop(0, jblocks, jbody,
                                       (neg, zer, neg, zer), unroll=4)
        rowc = lax.broadcasted_iota(jnp.uint32, (b, 1), 0)
        rb = rowc * np.uint32(n) + su * bn
        ctr_wa = rb + lax.convert_element_type(ia, jnp.uint32)
        ctr_wb = rb + lax.convert_element_type(ib, jnp.uint32)
        g_wa = _gumbel_from_bits(_threefry_bits(ka1, ka2, ctr_wa))
        g_wb = _gumbel_from_bits(_threefry_bits(kb1, kb2, ctr_wb))
        ia_ref[...] = ia.reshape(1, b, 1)
        lwa_ref[...] = (ma - g_wa).reshape(1, b, 1)
        ib_ref[...] = ib.reshape(1, b, 1)
        lwb_ref[...] = (mb - g_wb).reshape(1, b, 1)

    outspec = pl.BlockSpec((1, b, 1), lambda s: (s, 0, 0))
    return pl.pallas_call(
        body,
        grid=(s_count,),
        in_specs=[pl.BlockSpec((jblocks, b, cb), lambda s: (0, 0, 0)),
                  pl.BlockSpec((jblocks, b, cb), lambda s: (0, 0, 0))],
        out_specs=[outspec, outspec, outspec, outspec],
        out_shape=[jax.ShapeDtypeStruct((s_count, b, 1), jnp.int32),
                   jax.ShapeDtypeStruct((s_count, b, 1), jnp.float32),
                   jax.ShapeDtypeStruct((s_count, b, 1), jnp.int32),
                   jax.ShapeDtypeStruct((s_count, b, 1), jnp.float32)],
        scratch_shapes=[pltpu.VMEM((b, cb), jnp.uint32)],
        compiler_params=pltpu.CompilerParams(
            dimension_semantics=("parallel",)),
        interpret=interpret,
    )


_NW = 32
_ROWS_PER_W = _B // _NW
_L = 16


def _phase2_body(ia_hbm, ib_hbm, lwa_hbm, lwb_hbm, out_hbm,
                 iav, ibv, lav, lbv, rv, vv, row):
    wid = lax.axis_index("s") * 2 + lax.axis_index("c")
    for t in range(_ROWS_PER_W):
        brow = wid * _ROWS_PER_W + t
        pltpu.sync_copy(ia_hbm.at[brow], iav)
        pltpu.sync_copy(ib_hbm.at[brow], ibv)
        pltpu.sync_copy(lwa_hbm.at[brow], lav)
        pltpu.sync_copy(lwb_hbm.at[brow], lbv)

        nv = jnp.zeros((_L,), jnp.float32)
        for c in range(_SPAD // _L):
            sl = pl.ds(c * _L, _L)
            pa = jnp.exp(lav[sl]) - jnp.float32(1e-12)
            pb = jnp.exp(lbv[sl]) - jnp.float32(1e-12)
            v = pa * pb
            if (c + 1) * _L > _S:
                lane = lax.broadcasted_iota(jnp.int32, (_L,), 0) + np.int32(c * _L)
                v = jnp.where(lane < np.int32(_S), v, jnp.float32(0.0))
            r = lax.rem(iav[sl] + ibv[sl], np.int32(_N))
            rv[sl] = r
            vv[sl] = v
            nv = nv + v
        norm = nv[0]
        for k in range(1, _L):
            norm = norm + nv[k]
        norm = jnp.maximum(norm, jnp.float32(1e-12))
        for c in range(_SPAD // _L):
            sl = pl.ds(c * _L, _L)
            vv[sl] = vv[sl] / norm

        def zbody(i, _):
            off = pl.multiple_of(i * _L, _L)
            row[pl.ds(off, _L)] = jnp.zeros((_L,), jnp.float32)
            return 0

        lax.fori_loop(0, _N // _L, zbody, 0)

        iota16 = lax.broadcasted_iota(jnp.int32, (_L,), 0)
        for c in range(_SPAD // _L):
            sl = pl.ds(c * _L, _L)
            r16 = rv[sl]
            v16 = vv[sl]
            for k in range(_L):
                if c * _L + k < _S:
                    plsc.addupdate_scatter(row, [r16], v16,
                                           mask=iota16 == np.int32(k))

        pltpu.sync_copy(row, out_hbm.at[brow])


@functools.cache
def _phase2():
    return pl.kernel(
        _phase2_body,
        mesh=plsc.VectorSubcoreMesh(core_axis_name="c", subcore_axis_name="s"),
        out_type=jax.ShapeDtypeStruct((_B, _N), jnp.float32),
        compiler_params=pltpu.CompilerParams(needs_layout_passes=False),
        scratch_types=[
            pltpu.VMEM((_SPAD,), jnp.int32),
            pltpu.VMEM((_SPAD,), jnp.int32),
            pltpu.VMEM((_SPAD,), jnp.float32),
            pltpu.VMEM((_SPAD,), jnp.float32),
            pltpu.VMEM((_SPAD,), jnp.int32),
            pltpu.VMEM((_SPAD,), jnp.float32),
            pltpu.VMEM((_N,), jnp.float32),
        ],
    )


def _to_blocks(logits):
    lp = jnp.pad(logits, ((0, 0), (0, _NPAD - _N)),
                 constant_values=-np.inf)
    return lp.reshape(_B, _J, _CB).transpose(1, 0, 2)


def kernel(probs_a, probs_b):
    la = jnp.log(probs_a + 1e-12)
    lb = jnp.log(probs_b + 1e-12)
    la3 = _to_blocks(la)
    lb3 = _to_blocks(lb)
    sampler = _make_sampler((_KA, _KB), _S, _B, _N, _CB, _J)
    ia, lwa, ib, lwb = sampler(la3, lb3)
    pad = ((0, 0), (0, _SPAD - _S))
    ia = jnp.pad(ia[:, :, 0].T, pad)
    ib = jnp.pad(ib[:, :, 0].T, pad)
    lwa = jnp.pad(lwa[:, :, 0].T, pad)
    lwb = jnp.pad(lwb[:, :, 0].T, pad)
    out = _phase2()(ia, ib, lwa, lwb)
    return out


# speedup vs baseline: 1.0765x; 1.0038x over previous
import functools

import numpy as np
import jax
import jax.numpy as jnp
from jax import lax
from jax.experimental import pallas as pl
from jax.experimental.pallas import tpu as pltpu
from jax.experimental.pallas import tpu_sc as plsc

_N = 100000
_S = 100
_B = 64
_CB = 1024
_J = 98
_NPAD = _J * _CB
_SPAD = 112

_KA = (np.uint32(1832780943), np.uint32(270669613))
_KB = (np.uint32(64467757), np.uint32(2916123636))

_TINY = np.float32(np.finfo(np.float32).tiny)


def _rotl(x, r):
    return (x << r) | (x >> (32 - r))


def _threefry_mix(x0, x1, rots):
    for r in rots:
        x0 = x0 + x1
        x1 = _rotl(x1, r)
        x1 = x1 ^ x0
    return x0, x1


def _threefry_bits(k1, k2, lo, key_folded=False):
    ks2 = np.uint32(k1 ^ k2 ^ np.uint32(0x1BD11BDA))
    r0 = (13, 15, 26, 6)
    r1 = (17, 29, 16, 24)
    x0 = jnp.full_like(lo, k1)
    x1 = lo if key_folded else lo + k2
    x0, x1 = _threefry_mix(x0, x1, r0)
    x0 = x0 + k2
    x1 = x1 + np.uint32(ks2 + np.uint32(1))
    x0, x1 = _threefry_mix(x0, x1, r1)
    x0 = x0 + ks2
    x1 = x1 + np.uint32(k1 + np.uint32(2))
    x0, x1 = _threefry_mix(x0, x1, r0)
    x0 = x0 + k1
    x1 = x1 + np.uint32(k2 + np.uint32(3))
    x0, x1 = _threefry_mix(x0, x1, r1)
    x0 = x0 + k2
    x1 = x1 + np.uint32(ks2 + np.uint32(4))
    x0, x1 = _threefry_mix(x0, x1, r0)
    x0 = x0 + ks2
    x1 = x1 + np.uint32(k1 + np.uint32(5))
    return x0 ^ x1


def _gumbel_from_bits(bits, exact=True):
    fb = lax.bitcast_convert_type(
        (bits >> np.uint32(9)) | np.uint32(0x3F800000), jnp.float32)
    f = fb - jnp.float32(1.0)
    if exact:
        f = jnp.maximum(_TINY, f * jnp.float32(1.0) + _TINY)
    return -jnp.log(-jnp.log(f))


def _make_sampler(keys, s_count, b, n, cb, jblocks, interpret=False):
    (ka1, ka2), (kb1, kb2) = keys
    bn = np.uint32(b * n)

    def argstep(val, j, ju, lane_i, vbest, ibest, cbv):
        m = jnp.max(val, axis=1, keepdims=True)
        amask = val == m
        col = jnp.min(jnp.where(amask, lane_i, np.int32(2 ** 30)),
                      axis=1, keepdims=True)
        gidx = col + lax.convert_element_type(ju * np.uint32(cb), jnp.int32)
        upd = m > vbest
        return jnp.where(upd, m, vbest), jnp.where(upd, gidx, ibest)

    def body(la_ref, lb_ref, ia_ref, lwa_ref, ib_ref, lwb_ref, base_scr):
        s = pl.program_id(0)
        su = lax.convert_element_type(s, jnp.uint32)
        lane = lax.broadcasted_iota(jnp.uint32, (b, cb), 1)
        rowi = lax.broadcasted_iota(jnp.uint32, (b, cb), 0)
        base_scr[...] = rowi * np.uint32(n) + lane + su * bn

        def jbody(j, carry):
            va, ia, vb, ib = carry
            ju = lax.convert_element_type(j, jnp.uint32)
            ctr = base_scr[...] + ju * np.uint32(cb)
            lane_i = lax.broadcasted_iota(jnp.int32, (b, cb), 1)
            val_a = _gumbel_from_bits(
                _threefry_bits(ka1, ka2, ctr), exact=False) + la_ref[j]
            val_b = _gumbel_from_bits(
                _threefry_bits(kb1, kb2, ctr), exact=False) + lb_ref[j]
            va, ia = argstep(val_a, j, ju, lane_i, va, ia, cb)
            vb, ib = argstep(val_b, j, ju, lane_i, vb, ib, cb)
            return va, ia, vb, ib

        neg = jnp.full((b, 1), -jnp.inf, jnp.float32)
        zer = jnp.zeros((b, 1), jnp.int32)
        ma, ia, mb, ib = lax.fori_loop(0, jblocks, jbody,
                                       (neg, zer, neg, zer), unroll=7)
        rowc = lax.broadcasted_iota(jnp.uint32, (b, 1), 0)
        rb = rowc * np.uint32(n) + su * bn
        ctr_wa = rb + lax.convert_element_type(ia, jnp.uint32)
        ctr_wb = rb + lax.convert_element_type(ib, jnp.uint32)
        g_wa = _gumbel_from_bits(_threefry_bits(ka1, ka2, ctr_wa))
        g_wb = _gumbel_from_bits(_threefry_bits(kb1, kb2, ctr_wb))
        ia_ref[...] = ia.reshape(1, b, 1)
        lwa_ref[...] = (ma - g_wa).reshape(1, b, 1)
        ib_ref[...] = ib.reshape(1, b, 1)
        lwb_ref[...] = (mb - g_wb).reshape(1, b, 1)

    outspec = pl.BlockSpec((1, b, 1), lambda s: (s, 0, 0))
    return pl.pallas_call(
        body,
        grid=(s_count,),
        in_specs=[pl.BlockSpec((jblocks, b, cb), lambda s: (0, 0, 0)),
                  pl.BlockSpec((jblocks, b, cb), lambda s: (0, 0, 0))],
        out_specs=[outspec, outspec, outspec, outspec],
        out_shape=[jax.ShapeDtypeStruct((s_count, b, 1), jnp.int32),
                   jax.ShapeDtypeStruct((s_count, b, 1), jnp.float32),
                   jax.ShapeDtypeStruct((s_count, b, 1), jnp.int32),
                   jax.ShapeDtypeStruct((s_count, b, 1), jnp.float32)],
        scratch_shapes=[pltpu.VMEM((b, cb), jnp.uint32)],
        compiler_params=pltpu.CompilerParams(
            dimension_semantics=("parallel",)),
        interpret=interpret,
    )


_NW = 32
_ROWS_PER_W = _B // _NW
_L = 16


def _phase2_body(ia_hbm, ib_hbm, lwa_hbm, lwb_hbm, out_hbm,
                 iav, ibv, lav, lbv, rv, vv, row):
    wid = lax.axis_index("s") * 2 + lax.axis_index("c")
    for t in range(_ROWS_PER_W):
        brow = wid * _ROWS_PER_W + t
        pltpu.sync_copy(ia_hbm.at[brow], iav)
        pltpu.sync_copy(ib_hbm.at[brow], ibv)
        pltpu.sync_copy(lwa_hbm.at[brow], lav)
        pltpu.sync_copy(lwb_hbm.at[brow], lbv)

        nv = jnp.zeros((_L,), jnp.float32)
        for c in range(_SPAD // _L):
            sl = pl.ds(c * _L, _L)
            pa = jnp.exp(lav[sl]) - jnp.float32(1e-12)
            pb = jnp.exp(lbv[sl]) - jnp.float32(1e-12)
            v = pa * pb
            if (c + 1) * _L > _S:
                lane = lax.broadcasted_iota(jnp.int32, (_L,), 0) + np.int32(c * _L)
                v = jnp.where(lane < np.int32(_S), v, jnp.float32(0.0))
            r = lax.rem(iav[sl] + ibv[sl], np.int32(_N))
            rv[sl] = r
            vv[sl] = v
            nv = nv + v
        norm = nv[0]
        for k in range(1, _L):
            norm = norm + nv[k]
        norm = jnp.maximum(norm, jnp.float32(1e-12))
        for c in range(_SPAD // _L):
            sl = pl.ds(c * _L, _L)
            vv[sl] = vv[sl] / norm

        def zbody(i, _):
            off = pl.multiple_of(i * _L, _L)
            row[pl.ds(off, _L)] = jnp.zeros((_L,), jnp.float32)
            return 0

        lax.fori_loop(0, _N // _L, zbody, 0)

        iota16 = lax.broadcasted_iota(jnp.int32, (_L,), 0)
        for c in range(_SPAD // _L):
            sl = pl.ds(c * _L, _L)
            r16 = rv[sl]
            v16 = vv[sl]
            for k in range(_L):
                if c * _L + k < _S:
                    plsc.addupdate_scatter(row, [r16], v16,
                                           mask=iota16 == np.int32(k))

        pltpu.sync_copy(row, out_hbm.at[brow])


@functools.cache
def _phase2():
    return pl.kernel(
        _phase2_body,
        mesh=plsc.VectorSubcoreMesh(core_axis_name="c", subcore_axis_name="s"),
        out_type=jax.ShapeDtypeStruct((_B, _N), jnp.float32),
        compiler_params=pltpu.CompilerParams(needs_layout_passes=False),
        scratch_types=[
            pltpu.VMEM((_SPAD,), jnp.int32),
            pltpu.VMEM((_SPAD,), jnp.int32),
            pltpu.VMEM((_SPAD,), jnp.float32),
            pltpu.VMEM((_SPAD,), jnp.float32),
            pltpu.VMEM((_SPAD,), jnp.int32),
            pltpu.VMEM((_SPAD,), jnp.float32),
            pltpu.VMEM((_N,), jnp.float32),
        ],
    )


def _to_blocks(logits):
    lp = jnp.pad(logits, ((0, 0), (0, _NPAD - _N)),
                 constant_values=-np.inf)
    return lp.reshape(_B, _J, _CB).transpose(1, 0, 2)


def kernel(probs_a, probs_b):
    la = jnp.log(probs_a + 1e-12)
    lb = jnp.log(probs_b + 1e-12)
    la3 = _to_blocks(la)
    lb3 = _to_blocks(lb)
    sampler = _make_sampler((_KA, _KB), _S, _B, _N, _CB, _J)
    ia, lwa, ib, lwb = sampler(la3, lb3)
    pad = ((0, 0), (0, _SPAD - _S))
    ia = jnp.pad(ia[:, :, 0].T, pad)
    ib = jnp.pad(ib[:, :, 0].T, pad)
    lwa = jnp.pad(lwa[:, :, 0].T, pad)
    lwb = jnp.pad(lwb[:, :, 0].T, pad)
    out = _phase2()(ia, ib, lwa, lwb)
    return out


# speedup vs baseline: 1.0768x; 1.0003x over previous
import functools

import numpy as np
import jax
import jax.numpy as jnp
from jax import lax
from jax.experimental import pallas as pl
from jax.experimental.pallas import tpu as pltpu
from jax.experimental.pallas import tpu_sc as plsc

_N = 100000
_S = 100
_B = 64
_CB = 1024
_J = 98
_NPAD = _J * _CB
_SPAD = 112

_KA = (np.uint32(1832780943), np.uint32(270669613))
_KB = (np.uint32(64467757), np.uint32(2916123636))

_TINY = np.float32(np.finfo(np.float32).tiny)


def _rotl(x, r):
    return (x << r) | (x >> (32 - r))


def _threefry_mix(x0, x1, rots):
    for r in rots:
        x0 = x0 + x1
        x1 = _rotl(x1, r)
        x1 = x1 ^ x0
    return x0, x1


def _threefry_bits(k1, k2, lo, key_folded=False):
    ks2 = np.uint32(k1 ^ k2 ^ np.uint32(0x1BD11BDA))
    r0 = (13, 15, 26, 6)
    r1 = (17, 29, 16, 24)
    x0 = jnp.full_like(lo, k1)
    x1 = lo if key_folded else lo + k2
    x0, x1 = _threefry_mix(x0, x1, r0)
    x0 = x0 + k2
    x1 = x1 + np.uint32(ks2 + np.uint32(1))
    x0, x1 = _threefry_mix(x0, x1, r1)
    x0 = x0 + ks2
    x1 = x1 + np.uint32(k1 + np.uint32(2))
    x0, x1 = _threefry_mix(x0, x1, r0)
    x0 = x0 + k1
    x1 = x1 + np.uint32(k2 + np.uint32(3))
    x0, x1 = _threefry_mix(x0, x1, r1)
    x0 = x0 + k2
    x1 = x1 + np.uint32(ks2 + np.uint32(4))
    x0, x1 = _threefry_mix(x0, x1, r0)
    x0 = x0 + ks2
    x1 = x1 + np.uint32(k1 + np.uint32(5))
    return x0 ^ x1


def _gumbel_from_bits(bits, exact=True):
    fb = lax.bitcast_convert_type(
        (bits >> np.uint32(9)) | np.uint32(0x3F800000), jnp.float32)
    f = fb - jnp.float32(1.0)
    if exact:
        f = jnp.maximum(_TINY, f * jnp.float32(1.0) + _TINY)
    return -jnp.log(-jnp.log(f))


def _make_sampler(keys, s_count, b, n, cb, jblocks, interpret=False):
    (ka1, ka2), (kb1, kb2) = keys
    bn = np.uint32(b * n)

    def argstep(val, j, ju, lane_i, vbest, ibest, cbv):
        m = jnp.max(val, axis=1, keepdims=True)
        amask = val == m
        col = jnp.min(jnp.where(amask, lane_i, np.int32(2 ** 30)),
                      axis=1, keepdims=True)
        gidx = col + lax.convert_element_type(ju * np.uint32(cb), jnp.int32)
        upd = m > vbest
        return jnp.where(upd, m, vbest), jnp.where(upd, gidx, ibest)

    def body(la_ref, lb_ref, ia_ref, lwa_ref, ib_ref, lwb_ref, base_scr):
        s = pl.program_id(0)
        su = lax.convert_element_type(s, jnp.uint32)
        lane = lax.broadcasted_iota(jnp.uint32, (b, cb), 1)
        rowi = lax.broadcasted_iota(jnp.uint32, (b, cb), 0)
        base_scr[...] = rowi * np.uint32(n) + lane + su * bn

        def jbody(j, carry):
            va, ia, vb, ib = carry
            ju = lax.convert_element_type(j, jnp.uint32)
            ctr = base_scr[...] + ju * np.uint32(cb)
            lane_i = lax.broadcasted_iota(jnp.int32, (b, cb), 1)
            val_a = _gumbel_from_bits(
                _threefry_bits(ka1, ka2, ctr), exact=False) + la_ref[j]
            val_b = _gumbel_from_bits(
                _threefry_bits(kb1, kb2, ctr), exact=False) + lb_ref[j]
            va, ia = argstep(val_a, j, ju, lane_i, va, ia, cb)
            vb, ib = argstep(val_b, j, ju, lane_i, vb, ib, cb)
            return va, ia, vb, ib

        neg = jnp.full((b, 1), -jnp.inf, jnp.float32)
        zer = jnp.zeros((b, 1), jnp.int32)
        ma, ia, mb, ib = lax.fori_loop(0, jblocks, jbody,
                                       (neg, zer, neg, zer), unroll=14)
        rowc = lax.broadcasted_iota(jnp.uint32, (b, 1), 0)
        rb = rowc * np.uint32(n) + su * bn
        ctr_wa = rb + lax.convert_element_type(ia, jnp.uint32)
        ctr_wb = rb + lax.convert_element_type(ib, jnp.uint32)
        g_wa = _gumbel_from_bits(_threefry_bits(ka1, ka2, ctr_wa))
        g_wb = _gumbel_from_bits(_threefry_bits(kb1, kb2, ctr_wb))
        ia_ref[...] = ia.reshape(1, b, 1)
        lwa_ref[...] = (ma - g_wa).reshape(1, b, 1)
        ib_ref[...] = ib.reshape(1, b, 1)
        lwb_ref[...] = (mb - g_wb).reshape(1, b, 1)

    outspec = pl.BlockSpec((1, b, 1), lambda s: (s, 0, 0))
    return pl.pallas_call(
        body,
        grid=(s_count,),
        in_specs=[pl.BlockSpec((jblocks, b, cb), lambda s: (0, 0, 0)),
                  pl.BlockSpec((jblocks, b, cb), lambda s: (0, 0, 0))],
        out_specs=[outspec, outspec, outspec, outspec],
        out_shape=[jax.ShapeDtypeStruct((s_count, b, 1), jnp.int32),
                   jax.ShapeDtypeStruct((s_count, b, 1), jnp.float32),
                   jax.ShapeDtypeStruct((s_count, b, 1), jnp.int32),
                   jax.ShapeDtypeStruct((s_count, b, 1), jnp.float32)],
        scratch_shapes=[pltpu.VMEM((b, cb), jnp.uint32)],
        compiler_params=pltpu.CompilerParams(
            dimension_semantics=("parallel",)),
        interpret=interpret,
    )


_NW = 32
_ROWS_PER_W = _B // _NW
_L = 16


def _phase2_body(ia_hbm, ib_hbm, lwa_hbm, lwb_hbm, out_hbm,
                 iav, ibv, lav, lbv, rv, vv, row):
    wid = lax.axis_index("s") * 2 + lax.axis_index("c")
    for t in range(_ROWS_PER_W):
        brow = wid * _ROWS_PER_W + t
        pltpu.sync_copy(ia_hbm.at[brow], iav)
        pltpu.sync_copy(ib_hbm.at[brow], ibv)
        pltpu.sync_copy(lwa_hbm.at[brow], lav)
        pltpu.sync_copy(lwb_hbm.at[brow], lbv)

        nv = jnp.zeros((_L,), jnp.float32)
        for c in range(_SPAD // _L):
            sl = pl.ds(c * _L, _L)
            pa = jnp.exp(lav[sl]) - jnp.float32(1e-12)
            pb = jnp.exp(lbv[sl]) - jnp.float32(1e-12)
            v = pa * pb
            if (c + 1) * _L > _S:
                lane = lax.broadcasted_iota(jnp.int32, (_L,), 0) + np.int32(c * _L)
                v = jnp.where(lane < np.int32(_S), v, jnp.float32(0.0))
            r = lax.rem(iav[sl] + ibv[sl], np.int32(_N))
            rv[sl] = r
            vv[sl] = v
            nv = nv + v
        norm = nv[0]
        for k in range(1, _L):
            norm = norm + nv[k]
        norm = jnp.maximum(norm, jnp.float32(1e-12))
        for c in range(_SPAD // _L):
            sl = pl.ds(c * _L, _L)
            vv[sl] = vv[sl] / norm

        def zbody(i, _):
            off = pl.multiple_of(i * _L, _L)
            row[pl.ds(off, _L)] = jnp.zeros((_L,), jnp.float32)
            return 0

        lax.fori_loop(0, _N // _L, zbody, 0)

        iota16 = lax.broadcasted_iota(jnp.int32, (_L,), 0)
        for c in range(_SPAD // _L):
            sl = pl.ds(c * _L, _L)
            r16 = rv[sl]
            v16 = vv[sl]
            for k in range(_L):
                if c * _L + k < _S:
                    plsc.addupdate_scatter(row, [r16], v16,
                                           mask=iota16 == np.int32(k))

        pltpu.sync_copy(row, out_hbm.at[brow])


@functools.cache
def _phase2():
    return pl.kernel(
        _phase2_body,
        mesh=plsc.VectorSubcoreMesh(core_axis_name="c", subcore_axis_name="s"),
        out_type=jax.ShapeDtypeStruct((_B, _N), jnp.float32),
        compiler_params=pltpu.CompilerParams(needs_layout_passes=False),
        scratch_types=[
            pltpu.VMEM((_SPAD,), jnp.int32),
            pltpu.VMEM((_SPAD,), jnp.int32),
            pltpu.VMEM((_SPAD,), jnp.float32),
            pltpu.VMEM((_SPAD,), jnp.float32),
            pltpu.VMEM((_SPAD,), jnp.int32),
            pltpu.VMEM((_SPAD,), jnp.float32),
            pltpu.VMEM((_N,), jnp.float32),
        ],
    )


def _to_blocks(logits):
    lp = jnp.pad(logits, ((0, 0), (0, _NPAD - _N)),
                 constant_values=-np.inf)
    return lp.reshape(_B, _J, _CB).transpose(1, 0, 2)


def kernel(probs_a, probs_b):
    la = jnp.log(probs_a + 1e-12)
    lb = jnp.log(probs_b + 1e-12)
    la3 = _to_blocks(la)
    lb3 = _to_blocks(lb)
    sampler = _make_sampler((_KA, _KB), _S, _B, _N, _CB, _J)
    ia, lwa, ib, lwb = sampler(la3, lb3)
    pad = ((0, 0), (0, _SPAD - _S))
    ia = jnp.pad(ia[:, :, 0].T, pad)
    ib = jnp.pad(ib[:, :, 0].T, pad)
    lwa = jnp.pad(lwa[:, :, 0].T, pad)
    lwb = jnp.pad(lwb[:, :, 0].T, pad)
    out = _phase2()(ia, ib, lwa, lwb)
    return out
